# baseline (speedup 1.0000x reference)
import functools

import jax
import jax.numpy as jnp
from jax import lax
from jax.experimental import pallas as pl
from jax.experimental.pallas import tpu as pltpu
from jax.experimental.pallas import tpu_sc as plsc

N = 10000
E = 320000
IN = 128
HID = 128
OUT = 64
R = 8

NC = 2
NS = 16
L = 16

CE = 128
CHB = 8
EP = 327680
NCH = EP // CE
RW = NCH // (NC * NS)
NB = RW // CHB
EW = RW * CE
CCS = NCH // NS
NR = N * R
NRP = 80384
SEG_SLICE = NRP // NS
NP = 10240
ZROWS = 64

_mesh = plsc.VectorSubcoreMesh(
    core_axis_name="c", subcore_axis_name="s", num_cores=NC, num_subcores=NS
)
_sc_params = pltpu.CompilerParams(
    needs_layout_passes=False, use_tc_tiling_on_sc=False
)


@functools.partial(
    pl.kernel,
    out_type=jax.ShapeDtypeStruct((NCH, CE), jnp.float32),
    mesh=_mesh,
    scratch_types=[
        pltpu.VMEM((CHB, CE), jnp.int32),
        pltpu.VMEM((CHB, CE), jnp.float32),
        pltpu.VMEM((CE,), jnp.float32),
        pltpu.VMEM((SEG_SLICE,), jnp.float32),
        pltpu.VMEM_SHARED((NRP,), jnp.float32),
        pltpu.SemaphoreType.DMA,
    ],
    compiler_params=_sc_params,
)
def _counts_weights(seg_hbm, w_hbm, ssta, wsta, ones_v, acc_v, cnt_sh, sem):
    c = lax.axis_index("c")
    s = lax.axis_index("s")
    zero16 = jnp.zeros((L,), jnp.float32)
    one16 = jnp.full((L,), 1.0, jnp.float32)

    def zbody(i, _):
        off = pl.multiple_of(i * L, L)
        acc_v[pl.ds(off, L)] = zero16
        return _

    lax.fori_loop(0, SEG_SLICE // L, zbody, None)
    for k in range(CE // L):
        ones_v[pl.ds(k * L, L)] = one16
    off = s * SEG_SLICE
    pltpu.sync_copy(acc_v, cnt_sh.at[pl.ds(off, SEG_SLICE)])
    plsc.subcore_barrier()

    cbase = s * CCS

    def cbody(j, _):
        r0 = cbase + j * CHB
        pltpu.sync_copy(seg_hbm.at[pl.ds(r0, CHB)], ssta)
        for j2 in range(CHB):
            pltpu.async_copy(ones_v, cnt_sh.at[ssta.at[j2]], sem, add=True)
        for j2 in range(CHB):
            pltpu.make_async_copy(ones_v, cnt_sh.at[ssta.at[j2]], sem).wait()
        return _

    lax.fori_loop(0, CCS // CHB, cbody, None)
    plsc.subcore_barrier()

    pltpu.sync_copy(cnt_sh.at[pl.ds(off, SEG_SLICE)], acc_v)

    def ibody(i, _):
        o = pl.multiple_of(i * L, L)
        cnt = acc_v[pl.ds(o, L)]
        acc_v[pl.ds(o, L)] = jnp.where(
            cnt > 0.0, 1.0 / jnp.maximum(cnt, 1.0), 0.0
        )
        return _

    lax.fori_loop(0, SEG_SLICE // L, ibody, None)
    pltpu.sync_copy(acc_v, cnt_sh.at[pl.ds(off, SEG_SLICE)])
    plsc.subcore_barrier()

    wid = c * NS + s
    wbase = wid * RW

    def wbody(i, _):
        r0 = wbase + i * CHB
        pltpu.sync_copy(seg_hbm.at[pl.ds(r0, CHB)], ssta)
        for j2 in range(CHB):
            pltpu.async_copy(cnt_sh.at[ssta.at[j2]], wsta.at[j2], sem)
        for j2 in range(CHB):
            pltpu.make_async_copy(cnt_sh.at[ssta.at[j2]], wsta.at[j2], sem).wait()
        pltpu.sync_copy(wsta, w_hbm.at[pl.ds(r0, CHB)])
        return _

    lax.fori_loop(0, NB, wbody, None)


RW0_L1 = 120
RW1_L1 = NCH // NS - RW0_L1
H = CE // 2


@functools.partial(
    pl.kernel,
    out_type=jax.ShapeDtypeStruct((NC, NP, HID), jnp.float32),
    mesh=_mesh,
    scratch_types=[
        pltpu.VMEM((ZROWS, HID), jnp.float32),
        pltpu.VMEM((CHB, CE), jnp.int32),
        pltpu.VMEM((2 * CHB, H), jnp.int32),
        pltpu.VMEM((CHB, CE), jnp.float32),
        pltpu.VMEM((CE, HID), jnp.float32),
        pltpu.VMEM((CE, HID), jnp.float32),
        pltpu.VMEM_SHARED((NP, HID), jnp.float32),
        pltpu.SemaphoreType.DMA,
        pltpu.SemaphoreType.DMA,
        pltpu.SemaphoreType.DMA,
        pltpu.SemaphoreType.DMA,
        pltpu.SemaphoreType.DMA,
        pltpu.SemaphoreType.DMA,
        pltpu.SemaphoreType.DMA,
        pltpu.SemaphoreType.DMA,
    ],
    compiler_params=_sc_params,
)
def _layer_hid(table_hbm, gidx_hbm, dst64_hbm, w_hbm, part_hbm,
               zero_v, gsta, dsta64, wsta, rows_0, rows_1, agg_sh,
               sg0, sg1, sg2, sg3, ss0, ss1, ss2, ss3):
    KD = HID // L
    c = lax.axis_index("c")
    s = lax.axis_index("s")
    zero16 = jnp.zeros((L,), jnp.float32)

    def z1(i, _):
        for k in range(KD):
            zero_v[i, pl.ds(k * L, L)] = zero16
        return _

    lax.fori_loop(0, ZROWS, z1, None)
    zrows_per = NP // NS
    zbase = s * zrows_per

    def z2(j, _):
        pltpu.sync_copy(zero_v, agg_sh.at[pl.ds(zbase + j * ZROWS, ZROWS)])
        return _

    lax.fori_loop(0, zrows_per // ZROWS, z2, None)
    plsc.subcore_barrier()

    phys = (rows_0, rows_1)
    gsems = (sg0, sg1, sg2, sg3)
    ssems = (ss0, ss1, ss2, ss3)

    @pl.when(c == 0)
    def _core0():
        nbuf = 2
        wbase = s * RW0_L1

        def scale(rows, j2):
            def sbody(g, _2):
                go = pl.multiple_of(g * L, L)
                wv = wsta[j2, pl.ds(go, L)]
                for jl in range(L):
                    w = wv[jl]
                    for k in range(KD):
                        sl = pl.ds(k * L, L)
                        rows[go + jl, sl] = rows[go + jl, sl] * w
                return _2

            lax.fori_loop(0, CE // L, sbody, None)

        def fire_scatter(b, j2):
            pltpu.async_copy(
                phys[b].at[pl.ds(0, H)], agg_sh.at[dsta64.at[2 * j2]],
                ssems[b], add=True,
            )
            pltpu.async_copy(
                phys[b].at[pl.ds(H, H)], agg_sh.at[dsta64.at[2 * j2 + 1]],
                ssems[b], add=True,
            )

        def wait_scatter(b, j2):
            pltpu.make_async_copy(
                phys[b].at[pl.ds(0, H)], agg_sh.at[dsta64.at[2 * j2]], ssems[b]
            ).wait()
            pltpu.make_async_copy(
                phys[b].at[pl.ds(H, H)], agg_sh.at[dsta64.at[2 * j2 + 1]], ssems[b]
            ).wait()

        def bbody(bi, first):
            r0 = wbase + bi * CHB
            pltpu.sync_copy(gidx_hbm.at[pl.ds(r0, CHB)], gsta)
            pltpu.sync_copy(dst64_hbm.at[pl.ds(2 * r0, 2 * CHB)], dsta64)
            pltpu.sync_copy(w_hbm.at[pl.ds(r0, CHB)], wsta)
            pltpu.async_copy(table_hbm.at[gsta.at[0]], phys[0], gsems[0])
            for j2 in range(CHB):
                b = j2 % nbuf
                if j2 < CHB - 1:
                    nb2 = (j2 + 1) % nbuf
                    if j2 + 1 >= nbuf:
                        wait_scatter(nb2, j2 + 1 - nbuf)
                    pltpu.async_copy(
                        table_hbm.at[gsta.at[j2 + 1]], phys[nb2], gsems[nb2]
                    )
                pltpu.make_async_copy(
                    table_hbm.at[gsta.at[j2]], phys[b], gsems[b]
                ).wait()
                scale(phys[b], j2)
                fire_scatter(b, j2)
            for j2 in range(CHB - nbuf, CHB):
                wait_scatter(j2 % nbuf, j2)
            return first

        lax.fori_loop(0, RW0_L1 // CHB, bbody, 0)

    @pl.when(c != 0)
    def _core1():
        wbase = NS * RW0_L1 + s * RW1_L1
        nsub = 2 * CHB

        def buf_at(sl):
            p, hh = sl % 2, sl // 2
            return phys[p].at[pl.ds(hh * H, H)]

        def gidx_at(l):
            return gsta.at[l // 2, pl.ds((l % 2) * H, H)]

        def scale64(sl, l):
            p, hh = sl % 2, sl // 2
            rows = phys[p]
            rbase = hh * H
            j2, h2 = l // 2, l % 2

            def sbody(g, _2):
                go = pl.multiple_of(g * L, L)
                wv = wsta[j2, pl.ds(h2 * H + go, L)]
                for jl in range(L):
                    w = wv[jl]
                    for k in range(KD):
                        sl2 = pl.ds(k * L, L)
                        rows[rbase + go + jl, sl2] = rows[rbase + go + jl, sl2] * w
                return _2

            lax.fori_loop(0, H // L, sbody, None)

        def bbody(bi, first):
            r0 = wbase + bi * CHB
            pltpu.sync_copy(gidx_hbm.at[pl.ds(r0, CHB)], gsta)
            pltpu.sync_copy(dst64_hbm.at[pl.ds(2 * r0, 2 * CHB)], dsta64)
            pltpu.sync_copy(w_hbm.at[pl.ds(r0, CHB)], wsta)
            pltpu.async_copy(table_hbm.at[gidx_at(0)], buf_at(0), gsems[0])
            for l in range(nsub):
                sl = l % 4
                if l < nsub - 1:
                    nsl = (l + 1) % 4
                    if l + 1 >= 4:
                        pltpu.make_async_copy(
                            buf_at(nsl), agg_sh.at[dsta64.at[l + 1 - 4]],
                            ssems[nsl],
                        ).wait()
                    pltpu.async_copy(
                        table_hbm.at[gidx_at(l + 1)], buf_at(nsl), gsems[nsl]
                    )
                pltpu.make_async_copy(
                    table_hbm.at[gidx_at(l)], buf_at(sl), gsems[sl]
                ).wait()
                scale64(sl, l)
                pltpu.async_copy(
                    buf_at(sl), agg_sh.at[dsta64.at[l]], ssems[sl], add=True
                )
            for l in range(nsub - 4, nsub):
                sl = l % 4
                pltpu.make_async_copy(
                    buf_at(sl), agg_sh.at[dsta64.at[l]], ssems[sl]
                ).wait()
            return first

        lax.fori_loop(0, RW1_L1 // CHB, bbody, 0)

    plsc.subcore_barrier()
    drows = NP // NS
    pltpu.sync_copy(
        agg_sh.at[pl.ds(s * drows, drows)],
        part_hbm.at[c, pl.ds(s * drows, drows)],
    )


def _make_layer(d, nbuf, rw0):
    KD = d // L
    rw1 = NCH // NS - rw0
    nb0 = rw0 // CHB
    nb1 = rw1 // CHB

    @functools.partial(
        pl.kernel,
        out_type=jax.ShapeDtypeStruct((NC, NP, d), jnp.float32),
        mesh=_mesh,
        scratch_types=(
            [
                pltpu.VMEM((ZROWS, d), jnp.float32),
                pltpu.VMEM((CHB, CE), jnp.int32),
                pltpu.VMEM((CHB, CE), jnp.int32),
                pltpu.VMEM((CHB, CE), jnp.float32),
            ]
            + [pltpu.VMEM((CE, d), jnp.float32) for _ in range(nbuf)]
            + [pltpu.VMEM_SHARED((NP, d), jnp.float32)]
            + [pltpu.SemaphoreType.DMA for _ in range(2 * nbuf)]
        ),
        compiler_params=_sc_params,
    )
    def _layer(table_hbm, gidx_hbm, dst_hbm, w_hbm, part_hbm,
               zero_v, gsta, dsta, wsta, *rest):
        bufs = rest[:nbuf]
        agg_sh = rest[nbuf]
        gsems = rest[nbuf + 1:2 * nbuf + 1]
        ssems = rest[2 * nbuf + 1:]
        c = lax.axis_index("c")
        s = lax.axis_index("s")
        zero16 = jnp.zeros((L,), jnp.float32)

        def z1(i, _):
            for k in range(KD):
                zero_v[i, pl.ds(k * L, L)] = zero16
            return _

        lax.fori_loop(0, ZROWS, z1, None)

        zrows_per = NP // NS
        zbase = s * zrows_per

        def z2(j, _):
            pltpu.sync_copy(zero_v, agg_sh.at[pl.ds(zbase + j * ZROWS, ZROWS)])
            return _

        lax.fori_loop(0, zrows_per // ZROWS, z2, None)
        plsc.subcore_barrier()

        rw_c = jnp.where(c == 0, rw0, rw1)
        nb_c = jnp.where(c == 0, nb0, nb1)
        wbase = c * (NS * rw0) + s * rw_c

        def scale(rows, j2):
            def sbody(g, _2):
                go = pl.multiple_of(g * L, L)
                wv = wsta[j2, pl.ds(go, L)]
                for jl in range(L):
                    w = wv[jl]
                    for k in range(KD):
                        sl = pl.ds(k * L, L)
                        rows[go + jl, sl] = rows[go + jl, sl] * w
                return _2

            lax.fori_loop(0, CE // L, sbody, None)

        def bbody(bi, first):
            r0 = wbase + bi * CHB
            pltpu.sync_copy(gidx_hbm.at[pl.ds(r0, CHB)], gsta)
            pltpu.sync_copy(dst_hbm.at[pl.ds(r0, CHB)], dsta)
            pltpu.sync_copy(w_hbm.at[pl.ds(r0, CHB)], wsta)
            pltpu.async_copy(table_hbm.at[gsta.at[0]], bufs[0], gsems[0])
            for j2 in range(CHB):
                b = j2 % nbuf
                if j2 < CHB - 1:
                    nb2 = (j2 + 1) % nbuf
                    if j2 + 1 >= nbuf:
                        pltpu.make_async_copy(
                            bufs[nb2], agg_sh.at[dsta.at[j2 + 1 - nbuf]],
                            ssems[nb2],
                        ).wait()
                    pltpu.async_copy(
                        table_hbm.at[gsta.at[j2 + 1]], bufs[nb2], gsems[nb2]
                    )
                pltpu.make_async_copy(
                    table_hbm.at[gsta.at[j2]], bufs[b], gsems[b]
                ).wait()
                scale(bufs[b], j2)
                pltpu.async_copy(bufs[b], agg_sh.at[dsta.at[j2]], ssems[b], add=True)
            for j2 in range(CHB - nbuf, CHB):
                b = j2 % nbuf
                pltpu.make_async_copy(
                    bufs[b], agg_sh.at[dsta.at[j2]], ssems[b]
                ).wait()
            return first

        lax.fori_loop(0, nb_c, bbody, 0)
        plsc.subcore_barrier()

        drows = NP // NS
        pltpu.sync_copy(
            agg_sh.at[pl.ds(s * drows, drows)],
            part_hbm.at[c, pl.ds(s * drows, drows)],
        )

    return _layer


_layer_out = _make_layer(OUT, 4, 96)


BN = 2000


def _table_matmul(x, W):
    din = x.shape[1]
    dout = W.shape[2]

    def body(xr, wr, outr):
        outr[0] = jnp.dot(xr[...], wr[0], preferred_element_type=jnp.float32)

    return pl.pallas_call(
        body,
        grid=(R, N // BN),
        in_specs=[
            pl.BlockSpec((BN, din), lambda r, i: (i, 0)),
            pl.BlockSpec((1, din, dout), lambda r, i: (r, 0, 0)),
        ],
        out_specs=pl.BlockSpec((1, BN, dout), lambda r, i: (r, i, 0)),
        out_shape=jax.ShapeDtypeStruct((R, N, dout), jnp.float32),
    )(x, W)


def _combine(part, x, Wroot, b, act):
    din = x.shape[1]
    dout = Wroot.shape[1]

    def body(pr, xr, wr, br, outr):
        acc = pr[0] + pr[1] + jnp.dot(xr[...], wr[...], preferred_element_type=jnp.float32)
        outr[...] = act(acc + br[0])

    return pl.pallas_call(
        body,
        grid=(N // BN,),
        in_specs=[
            pl.BlockSpec((NC, BN, dout), lambda i: (0, i, 0)),
            pl.BlockSpec((BN, din), lambda i: (i, 0)),
            pl.BlockSpec((din, dout), lambda i: (0, 0)),
            pl.BlockSpec((1, dout), lambda i: (0, 0)),
        ],
        out_specs=pl.BlockSpec((BN, dout), lambda i: (i, 0)),
        out_shape=jax.ShapeDtypeStruct((N, dout), jnp.float32),
    )(part, x, Wroot, b.reshape(1, dout))


def kernel(x, edge_index, edge_type, W1_rel, W1_root, b1, W2_rel, W2_root, b2):
    src = edge_index[0].astype(jnp.int32)
    dst = edge_index[1].astype(jnp.int32)
    et = edge_type.astype(jnp.int32)
    pad = EP - E
    seg1 = jnp.concatenate([dst * R + et, jnp.full((pad,), NR, jnp.int32)]).reshape(NCH, CE)
    gid1 = jnp.concatenate([et * N + src, jnp.zeros((pad,), jnp.int32)]).reshape(NCH, CE)
    dst1 = jnp.concatenate([dst, jnp.full((pad,), N, jnp.int32)]).reshape(NCH, CE)

    w1 = _counts_weights(seg1)

    t1 = _table_matmul(x, W1_rel).reshape(R * N, HID)
    p1 = _layer_hid(t1, gid1, dst1.reshape(2 * NCH, CE // 2), w1)
    h = _combine(p1, x, W1_root, b1, lambda a: jnp.maximum(a, 0.0))

    t2 = _table_matmul(h, W2_rel).reshape(R * N, OUT)
    p2 = _layer_out(t2, gid1, dst1, w1)
    return _combine(p2, h, W2_root, b2, jax.nn.sigmoid)


# speedup vs baseline: 1.0054x; 1.0054x over previous
import functools

import jax
import jax.numpy as jnp
from jax import lax
from jax.experimental import pallas as pl
from jax.experimental.pallas import tpu as pltpu
from jax.experimental.pallas import tpu_sc as plsc

N = 10000
E = 320000
IN = 128
HID = 128
OUT = 64
R = 8

NC = 2
NS = 16
L = 16

CE = 128
CHB = 8
EP = 327680
NCH = EP // CE
RW = NCH // (NC * NS)
NB = RW // CHB
EW = RW * CE
CCS = NCH // NS
NR = N * R
NRP = 80384
SEG_SLICE = NRP // NS
NP = 10240
ZROWS = 64

_mesh = plsc.VectorSubcoreMesh(
    core_axis_name="c", subcore_axis_name="s", num_cores=NC, num_subcores=NS
)
_sc_params = pltpu.CompilerParams(
    needs_layout_passes=False, use_tc_tiling_on_sc=False
)


@functools.partial(
    pl.kernel,
    out_type=jax.ShapeDtypeStruct((NCH, CE), jnp.float32),
    mesh=_mesh,
    scratch_types=[
        pltpu.VMEM((CHB, CE), jnp.int32),
        pltpu.VMEM((CHB, CE), jnp.float32),
        pltpu.VMEM((CE,), jnp.float32),
        pltpu.VMEM((SEG_SLICE,), jnp.float32),
        pltpu.VMEM_SHARED((NRP,), jnp.float32),
        pltpu.SemaphoreType.DMA,
    ],
    compiler_params=_sc_params,
)
def _counts_weights(seg_hbm, w_hbm, ssta, wsta, ones_v, acc_v, cnt_sh, sem):
    c = lax.axis_index("c")
    s = lax.axis_index("s")
    zero16 = jnp.zeros((L,), jnp.float32)
    one16 = jnp.full((L,), 1.0, jnp.float32)

    def zbody(i, _):
        off = pl.multiple_of(i * L, L)
        acc_v[pl.ds(off, L)] = zero16
        return _

    lax.fori_loop(0, SEG_SLICE // L, zbody, None)
    for k in range(CE // L):
        ones_v[pl.ds(k * L, L)] = one16
    off = s * SEG_SLICE
    pltpu.sync_copy(acc_v, cnt_sh.at[pl.ds(off, SEG_SLICE)])
    plsc.subcore_barrier()

    cbase = s * CCS

    def cbody(j, _):
        r0 = cbase + j * CHB
        pltpu.sync_copy(seg_hbm.at[pl.ds(r0, CHB)], ssta)
        for j2 in range(CHB):
            pltpu.async_copy(ones_v, cnt_sh.at[ssta.at[j2]], sem, add=True)
        for j2 in range(CHB):
            pltpu.make_async_copy(ones_v, cnt_sh.at[ssta.at[j2]], sem).wait()
        return _

    lax.fori_loop(0, CCS // CHB, cbody, None)
    plsc.subcore_barrier()

    pltpu.sync_copy(cnt_sh.at[pl.ds(off, SEG_SLICE)], acc_v)

    def ibody(i, _):
        o = pl.multiple_of(i * L, L)
        cnt = acc_v[pl.ds(o, L)]
        acc_v[pl.ds(o, L)] = jnp.where(
            cnt > 0.0, 1.0 / jnp.maximum(cnt, 1.0), 0.0
        )
        return _

    lax.fori_loop(0, SEG_SLICE // L, ibody, None)
    pltpu.sync_copy(acc_v, cnt_sh.at[pl.ds(off, SEG_SLICE)])
    plsc.subcore_barrier()

    wid = c * NS + s
    wbase = wid * RW

    def wbody(i, _):
        r0 = wbase + i * CHB
        pltpu.sync_copy(seg_hbm.at[pl.ds(r0, CHB)], ssta)
        for j2 in range(CHB):
            pltpu.async_copy(cnt_sh.at[ssta.at[j2]], wsta.at[j2], sem)
        for j2 in range(CHB):
            pltpu.make_async_copy(cnt_sh.at[ssta.at[j2]], wsta.at[j2], sem).wait()
        pltpu.sync_copy(wsta, w_hbm.at[pl.ds(r0, CHB)])
        return _

    lax.fori_loop(0, NB, wbody, None)


RW0_L1 = 136
RW1_L1 = NCH // NS - RW0_L1
H = CE // 2


@functools.partial(
    pl.kernel,
    out_type=jax.ShapeDtypeStruct((NC, NP, HID), jnp.float32),
    mesh=_mesh,
    scratch_types=[
        pltpu.VMEM((ZROWS, HID), jnp.float32),
        pltpu.VMEM((CHB, CE), jnp.int32),
        pltpu.VMEM((2 * CHB, H), jnp.int32),
        pltpu.VMEM((CHB, CE), jnp.float32),
        pltpu.VMEM((CE, HID), jnp.float32),
        pltpu.VMEM((CE, HID), jnp.float32),
        pltpu.VMEM_SHARED((NP, HID), jnp.float32),
        pltpu.SemaphoreType.DMA,
        pltpu.SemaphoreType.DMA,
        pltpu.SemaphoreType.DMA,
        pltpu.SemaphoreType.DMA,
        pltpu.SemaphoreType.DMA,
        pltpu.SemaphoreType.DMA,
        pltpu.SemaphoreType.DMA,
        pltpu.SemaphoreType.DMA,
    ],
    compiler_params=_sc_params,
)
def _layer_hid(table_hbm, gidx_hbm, dst64_hbm, w_hbm, part_hbm,
               zero_v, gsta, dsta64, wsta, rows_0, rows_1, agg_sh,
               sg0, sg1, sg2, sg3, ss0, ss1, ss2, ss3):
    KD = HID // L
    c = lax.axis_index("c")
    s = lax.axis_index("s")
    zero16 = jnp.zeros((L,), jnp.float32)

    def z1(i, _):
        for k in range(KD):
            zero_v[i, pl.ds(k * L, L)] = zero16
        return _

    lax.fori_loop(0, ZROWS, z1, None)
    zrows_per = NP // NS
    zbase = s * zrows_per

    def z2(j, _):
        pltpu.sync_copy(zero_v, agg_sh.at[pl.ds(zbase + j * ZROWS, ZROWS)])
        return _

    lax.fori_loop(0, zrows_per // ZROWS, z2, None)
    plsc.subcore_barrier()

    phys = (rows_0, rows_1)
    gsems = (sg0, sg1, sg2, sg3)
    ssems = (ss0, ss1, ss2, ss3)

    @pl.when(c == 0)
    def _core0():
        nbuf = 2
        wbase = s * RW0_L1

        def scale(rows, j2):
            def sbody(g, _2):
                go = pl.multiple_of(g * L, L)
                wv = wsta[j2, pl.ds(go, L)]
                for jl in range(L):
                    w = wv[jl]
                    for k in range(KD):
                        sl = pl.ds(k * L, L)
                        rows[go + jl, sl] = rows[go + jl, sl] * w
                return _2

            lax.fori_loop(0, CE // L, sbody, None)

        def fire_scatter(b, j2):
            pltpu.async_copy(
                phys[b].at[pl.ds(0, H)], agg_sh.at[dsta64.at[2 * j2]],
                ssems[b], add=True,
            )
            pltpu.async_copy(
                phys[b].at[pl.ds(H, H)], agg_sh.at[dsta64.at[2 * j2 + 1]],
                ssems[b], add=True,
            )

        def wait_scatter(b, j2):
            pltpu.make_async_copy(
                phys[b].at[pl.ds(0, H)], agg_sh.at[dsta64.at[2 * j2]], ssems[b]
            ).wait()
            pltpu.make_async_copy(
                phys[b].at[pl.ds(H, H)], agg_sh.at[dsta64.at[2 * j2 + 1]], ssems[b]
            ).wait()

        def bbody(bi, first):
            r0 = wbase + bi * CHB
            pltpu.sync_copy(gidx_hbm.at[pl.ds(r0, CHB)], gsta)
            pltpu.sync_copy(dst64_hbm.at[pl.ds(2 * r0, 2 * CHB)], dsta64)
            pltpu.sync_copy(w_hbm.at[pl.ds(r0, CHB)], wsta)
            pltpu.async_copy(table_hbm.at[gsta.at[0]], phys[0], gsems[0])
            for j2 in range(CHB):
                b = j2 % nbuf
                if j2 < CHB - 1:
                    nb2 = (j2 + 1) % nbuf
                    if j2 + 1 >= nbuf:
                        wait_scatter(nb2, j2 + 1 - nbuf)
                    pltpu.async_copy(
                        table_hbm.at[gsta.at[j2 + 1]], phys[nb2], gsems[nb2]
                    )
                pltpu.make_async_copy(
                    table_hbm.at[gsta.at[j2]], phys[b], gsems[b]
                ).wait()
                scale(phys[b], j2)
                fire_scatter(b, j2)
            for j2 in range(CHB - nbuf, CHB):
                wait_scatter(j2 % nbuf, j2)
            return first

        lax.fori_loop(0, RW0_L1 // CHB, bbody, 0)

    @pl.when(c != 0)
    def _core1():
        wbase = NS * RW0_L1 + s * RW1_L1
        nsub = 2 * CHB

        def buf_at(sl):
            p, hh = sl % 2, sl // 2
            return phys[p].at[pl.ds(hh * H, H)]

        def gidx_at(l):
            return gsta.at[l // 2, pl.ds((l % 2) * H, H)]

        def scale64(sl, l):
            p, hh = sl % 2, sl // 2
            rows = phys[p]
            rbase = hh * H
            j2, h2 = l // 2, l % 2

            def sbody(g, _2):
                go = pl.multiple_of(g * L, L)
                wv = wsta[j2, pl.ds(h2 * H + go, L)]
                for jl in range(L):
                    w = wv[jl]
                    for k in range(KD):
                        sl2 = pl.ds(k * L, L)
                        rows[rbase + go + jl, sl2] = rows[rbase + go + jl, sl2] * w
                return _2

            lax.fori_loop(0, H // L, sbody, None)

        def bbody(bi, first):
            r0 = wbase + bi * CHB
            pltpu.sync_copy(gidx_hbm.at[pl.ds(r0, CHB)], gsta)
            pltpu.sync_copy(dst64_hbm.at[pl.ds(2 * r0, 2 * CHB)], dsta64)
            pltpu.sync_copy(w_hbm.at[pl.ds(r0, CHB)], wsta)
            pltpu.async_copy(table_hbm.at[gidx_at(0)], buf_at(0), gsems[0])
            for l in range(nsub):
                sl = l % 4
                if l < nsub - 1:
                    nsl = (l + 1) % 4
                    if l + 1 >= 4:
                        pltpu.make_async_copy(
                            buf_at(nsl), agg_sh.at[dsta64.at[l + 1 - 4]],
                            ssems[nsl],
                        ).wait()
                    pltpu.async_copy(
                        table_hbm.at[gidx_at(l + 1)], buf_at(nsl), gsems[nsl]
                    )
                pltpu.make_async_copy(
                    table_hbm.at[gidx_at(l)], buf_at(sl), gsems[sl]
                ).wait()
                scale64(sl, l)
                pltpu.async_copy(
                    buf_at(sl), agg_sh.at[dsta64.at[l]], ssems[sl], add=True
                )
            for l in range(nsub - 4, nsub):
                sl = l % 4
                pltpu.make_async_copy(
                    buf_at(sl), agg_sh.at[dsta64.at[l]], ssems[sl]
                ).wait()
            return first

        lax.fori_loop(0, RW1_L1 // CHB, bbody, 0)

    plsc.subcore_barrier()
    drows = NP // NS
    pltpu.sync_copy(
        agg_sh.at[pl.ds(s * drows, drows)],
        part_hbm.at[c, pl.ds(s * drows, drows)],
    )


def _make_layer(d, nbuf, rw0):
    KD = d // L
    rw1 = NCH // NS - rw0
    nb0 = rw0 // CHB
    nb1 = rw1 // CHB

    @functools.partial(
        pl.kernel,
        out_type=jax.ShapeDtypeStruct((NC, NP, d), jnp.float32),
        mesh=_mesh,
        scratch_types=(
            [
                pltpu.VMEM((ZROWS, d), jnp.float32),
                pltpu.VMEM((CHB, CE), jnp.int32),
                pltpu.VMEM((CHB, CE), jnp.int32),
                pltpu.VMEM((CHB, CE), jnp.float32),
            ]
            + [pltpu.VMEM((CE, d), jnp.float32) for _ in range(nbuf)]
            + [pltpu.VMEM_SHARED((NP, d), jnp.float32)]
            + [pltpu.SemaphoreType.DMA for _ in range(2 * nbuf)]
        ),
        compiler_params=_sc_params,
    )
    def _layer(table_hbm, gidx_hbm, dst_hbm, w_hbm, part_hbm,
               zero_v, gsta, dsta, wsta, *rest):
        bufs = rest[:nbuf]
        agg_sh = rest[nbuf]
        gsems = rest[nbuf + 1:2 * nbuf + 1]
        ssems = rest[2 * nbuf + 1:]
        c = lax.axis_index("c")
        s = lax.axis_index("s")
        zero16 = jnp.zeros((L,), jnp.float32)

        def z1(i, _):
            for k in range(KD):
                zero_v[i, pl.ds(k * L, L)] = zero16
            return _

        lax.fori_loop(0, ZROWS, z1, None)

        zrows_per = NP // NS
        zbase = s * zrows_per

        def z2(j, _):
            pltpu.sync_copy(zero_v, agg_sh.at[pl.ds(zbase + j * ZROWS, ZROWS)])
            return _

        lax.fori_loop(0, zrows_per // ZROWS, z2, None)
        plsc.subcore_barrier()

        rw_c = jnp.where(c == 0, rw0, rw1)
        nb_c = jnp.where(c == 0, nb0, nb1)
        wbase = c * (NS * rw0) + s * rw_c

        def scale(rows, j2):
            def sbody(g, _2):
                go = pl.multiple_of(g * L, L)
                wv = wsta[j2, pl.ds(go, L)]
                for jl in range(L):
                    w = wv[jl]
                    for k in range(KD):
                        sl = pl.ds(k * L, L)
                        rows[go + jl, sl] = rows[go + jl, sl] * w
                return _2

            lax.fori_loop(0, CE // L, sbody, None)

        def bbody(bi, first):
            r0 = wbase + bi * CHB
            pltpu.sync_copy(gidx_hbm.at[pl.ds(r0, CHB)], gsta)
            pltpu.sync_copy(dst_hbm.at[pl.ds(r0, CHB)], dsta)
            pltpu.sync_copy(w_hbm.at[pl.ds(r0, CHB)], wsta)
            pltpu.async_copy(table_hbm.at[gsta.at[0]], bufs[0], gsems[0])
            for j2 in range(CHB):
                b = j2 % nbuf
                if j2 < CHB - 1:
                    nb2 = (j2 + 1) % nbuf
                    if j2 + 1 >= nbuf:
                        pltpu.make_async_copy(
                            bufs[nb2], agg_sh.at[dsta.at[j2 + 1 - nbuf]],
                            ssems[nb2],
                        ).wait()
                    pltpu.async_copy(
                        table_hbm.at[gsta.at[j2 + 1]], bufs[nb2], gsems[nb2]
                    )
                pltpu.make_async_copy(
                    table_hbm.at[gsta.at[j2]], bufs[b], gsems[b]
                ).wait()
                scale(bufs[b], j2)
                pltpu.async_copy(bufs[b], agg_sh.at[dsta.at[j2]], ssems[b], add=True)
            for j2 in range(CHB - nbuf, CHB):
                b = j2 % nbuf
                pltpu.make_async_copy(
                    bufs[b], agg_sh.at[dsta.at[j2]], ssems[b]
                ).wait()
            return first

        lax.fori_loop(0, nb_c, bbody, 0)
        plsc.subcore_barrier()

        drows = NP // NS
        pltpu.sync_copy(
            agg_sh.at[pl.ds(s * drows, drows)],
            part_hbm.at[c, pl.ds(s * drows, drows)],
        )

    return _layer


_layer_out = _make_layer(OUT, 4, 96)


BN = 2000


def _table_matmul(x, W):
    din = x.shape[1]
    dout = W.shape[2]
    nb = N // BN

    def body(xr, wr, outr):
        outr[...] = jnp.dot(xr[...], wr[0], preferred_element_type=jnp.float32)

    return pl.pallas_call(
        body,
        grid=(R, nb),
        in_specs=[
            pl.BlockSpec((BN, din), lambda r, i: (i, 0)),
            pl.BlockSpec((1, din, dout), lambda r, i: (r, 0, 0)),
        ],
        out_specs=pl.BlockSpec((BN, dout), lambda r, i: (r * nb + i, 0)),
        out_shape=jax.ShapeDtypeStruct((R * N, dout), jnp.float32),
    )(x, W)


def _combine1_table2(part, x, W1_root, b1, W2_rel):
    nb = N // BN

    def body(pr, xr, wr, br, w2r, outh, outt):
        acc = pr[0] + pr[1] + jnp.dot(
            xr[...], wr[...], preferred_element_type=jnp.float32
        )
        hb = jnp.maximum(acc + br[0], 0.0)
        outh[...] = hb
        outt[...] = jnp.dot(hb, w2r[0], preferred_element_type=jnp.float32)

    return pl.pallas_call(
        body,
        grid=(R, nb),
        in_specs=[
            pl.BlockSpec((NC, BN, HID), lambda r, i: (0, i, 0)),
            pl.BlockSpec((BN, IN), lambda r, i: (i, 0)),
            pl.BlockSpec((IN, HID), lambda r, i: (0, 0)),
            pl.BlockSpec((1, HID), lambda r, i: (0, 0)),
            pl.BlockSpec((1, HID, OUT), lambda r, i: (r, 0, 0)),
        ],
        out_specs=[
            pl.BlockSpec((BN, HID), lambda r, i: (i, 0)),
            pl.BlockSpec((BN, OUT), lambda r, i: (r * nb + i, 0)),
        ],
        out_shape=[
            jax.ShapeDtypeStruct((N, HID), jnp.float32),
            jax.ShapeDtypeStruct((R * N, OUT), jnp.float32),
        ],
    )(part, x, W1_root, b1.reshape(1, HID), W2_rel)


def _combine(part, x, Wroot, b, act):
    din = x.shape[1]
    dout = Wroot.shape[1]

    def body(pr, xr, wr, br, outr):
        acc = pr[0] + pr[1] + jnp.dot(xr[...], wr[...], preferred_element_type=jnp.float32)
        outr[...] = act(acc + br[0])

    return pl.pallas_call(
        body,
        grid=(N // BN,),
        in_specs=[
            pl.BlockSpec((NC, BN, dout), lambda i: (0, i, 0)),
            pl.BlockSpec((BN, din), lambda i: (i, 0)),
            pl.BlockSpec((din, dout), lambda i: (0, 0)),
            pl.BlockSpec((1, dout), lambda i: (0, 0)),
        ],
        out_specs=pl.BlockSpec((BN, dout), lambda i: (i, 0)),
        out_shape=jax.ShapeDtypeStruct((N, dout), jnp.float32),
    )(part, x, Wroot, b.reshape(1, dout))


def kernel(x, edge_index, edge_type, W1_rel, W1_root, b1, W2_rel, W2_root, b2):
    src = edge_index[0].astype(jnp.int32)
    dst = edge_index[1].astype(jnp.int32)
    et = edge_type.astype(jnp.int32)
    pad = EP - E
    seg1 = jnp.concatenate([dst * R + et, jnp.full((pad,), NR, jnp.int32)]).reshape(NCH, CE)
    gid1 = jnp.concatenate([et * N + src, jnp.zeros((pad,), jnp.int32)]).reshape(NCH, CE)
    dst1 = jnp.concatenate([dst, jnp.full((pad,), N, jnp.int32)]).reshape(NCH, CE)

    w1 = _counts_weights(seg1)

    t1 = _table_matmul(x, W1_rel)
    p1 = _layer_hid(t1, gid1, dst1.reshape(2 * NCH, CE // 2), w1)
    h, t2 = _combine1_table2(p1, x, W1_root, b1, W2_rel)

    p2 = _layer_out(t2, gid1, dst1, w1)
    return _combine(p2, h, W2_root, b2, jax.nn.sigmoid)
